# Initial kernel scaffold; baseline (speedup 1.0000x reference)
import jax
import jax.numpy as jnp
from jax.experimental import pallas as pl


def kernel(music_librosa, pose_up, pose_down, label, params):
    raise NotImplementedError("write your pallas kernel here")


# speedup vs baseline: 7.1238x; 7.1238x over previous
import math

import numpy as np
import jax
import jax.numpy as jnp
from jax.experimental import pallas as pl
from jax.experimental.pallas import tpu as pltpu

D_MODEL = 512
D_STATE = 16
D_CONV = 4
D_INNER = 1024
DT_RANK = 32
N_HEAD = 8
HD = D_MODEL // N_HEAD
OUT_DIM = 4375
N_EXPERTS = 16
B = 32
TSRC = 29
SEG = 32
TP = 3 * SEG


def _build_padded_mask():
    ws, ts = 22, TSRC
    m = np.triu(np.ones((ts, ts), dtype=bool), 1)
    for i in range(ws, ts):
        m[i, :i - ws + 1] = True
    m87 = np.tile(m, (3, 3))
    mp = np.ones((TP, TP), dtype=bool)
    idx = np.concatenate([np.arange(ts), SEG + np.arange(ts), 2 * SEG + np.arange(ts)])
    mp[np.ix_(idx, idx)] = m87
    return mp


_MASK_NP = _build_padded_mask()


def _ln(x, g, b):
    mu = jnp.mean(x, axis=-1, keepdims=True)
    var = jnp.mean((x - mu) ** 2, axis=-1, keepdims=True)
    return (x - mu) * jax.lax.rsqrt(var + 1e-5) * g + b


def _dot(a, b):
    return jnp.dot(a, b, preferred_element_type=jnp.float32)


def _embed_kernel(pu_ref, pd_ref, up_tab, dn_tab, mus_ref,
                  m1w, m1b, m2w, m2b, o_ref):
    b = pl.program_id(0)
    o_ref[0] = jnp.zeros((TP, D_MODEL), jnp.float32)
    mus = mus_ref[0]
    h1 = _dot(mus, m1w[...]) + m1b[...]
    h2 = _dot(h1, m2w[...]) + m2b[...]
    o_ref[0, 2 * SEG:3 * SEG, :] = h2
    for l in range(TSRC):
        iu = pu_ref[b, l]
        idn = pd_ref[b, l]
        o_ref[0, l, :] = up_tab[iu, :]
        o_ref[0, SEG + l, :] = dn_tab[idn, :]


def _embed_call(pu_src, pd_src, up_tab, dn_tab, mus_p, m1, m2):
    whole = lambda a: pl.BlockSpec(a.shape, lambda b, pu, pd: (0,) * a.ndim)
    m1b = m1['b'][None, :]
    m2b = m2['b'][None, :]
    return pl.pallas_call(
        _embed_kernel,
        grid_spec=pltpu.PrefetchScalarGridSpec(
            num_scalar_prefetch=2,
            grid=(B,),
            in_specs=[
                whole(up_tab),
                whole(dn_tab),
                pl.BlockSpec((1, SEG, 35), lambda b, pu, pd: (b, 0, 0)),
                whole(m1['w']), whole(m1b), whole(m2['w']), whole(m2b),
            ],
            out_specs=pl.BlockSpec((1, TP, D_MODEL), lambda b, pu, pd: (b, 0, 0)),
        ),
        out_shape=jax.ShapeDtypeStruct((B, TP, D_MODEL), jnp.float32),
    )(pu_src, pd_src, up_tab, dn_tab, mus_p, m1['w'], m1b, m2['w'], m2b)


def _mamba_kernel(lbl_ref, x_ref, win, cw, cb, wxdt, wxB, wxC, wdt, bdt,
                  alog, dvec, wout, gg, bb, o_ref, h_ref, dA_ref, dBu_ref):
    x = x_ref[0, 0]
    xz = _dot(x, win[0, 0])
    xr = xz[:, :D_INNER]
    z = xz[:, D_INNER:]
    cwm = cw[0, 0]
    zrow = jnp.zeros((1, D_INNER), jnp.float32)
    s1 = jnp.concatenate([zrow, xr[:SEG - 1]], axis=0)
    s2 = jnp.concatenate([jnp.zeros((2, D_INNER), jnp.float32), xr[:SEG - 2]], axis=0)
    s3 = jnp.concatenate([jnp.zeros((3, D_INNER), jnp.float32), xr[:SEG - 3]], axis=0)
    conv = (cb[0, 0] + xr * cwm[3:4, :] + s1 * cwm[2:3, :]
            + s2 * cwm[1:2, :] + s3 * cwm[0:1, :])
    xc = jax.nn.silu(conv)
    dt = _dot(xc, wxdt[0, 0])
    Bm = _dot(xc, wxB[0, 0])
    Cm = _dot(xc, wxC[0, 0])
    delta = jax.nn.softplus(_dot(dt, wdt[0, 0]) + bdt[0, 0])
    A_T = -jnp.exp(alog[0, 0])
    dA_ref[...] = jnp.exp(delta[:, None, :] * A_T[None])
    dBu_ref[...] = Bm[:, :, None] * (delta * xc)[:, None, :]
    h_ref[...] = jnp.zeros((SEG, D_STATE, D_INNER), jnp.float32)

    def body(l, h):
        da = dA_ref[pl.ds(l, 1)][0]
        db = dBu_ref[pl.ds(l, 1)][0]
        h = da * h + db
        h_ref[pl.ds(l, 1)] = h[None]
        return h

    jax.lax.fori_loop(0, TSRC, body, jnp.zeros((D_STATE, D_INNER), jnp.float32))
    ys = jnp.sum(h_ref[...] * Cm[:, :, None], axis=1)
    y = ys + xc * dvec[0, 0]
    y = y * jax.nn.silu(z)
    out = x + _dot(y, wout[0, 0])
    o_ref[0, 0] = _ln(out, gg[0, 0], bb[0, 0])


def _mamba_call(feat4, mw, labels):
    names = ['win', 'cw', 'cb', 'wxdt', 'wxB', 'wxC', 'wdt', 'bdt',
             'alog', 'dvec', 'wout', 'g', 'b']
    arrays = [mw[n] for n in names]
    im_w = lambda i, b, lbl: (i, lbl[b], 0, 0)
    x_spec = pl.BlockSpec((1, 1, SEG, D_MODEL), lambda i, b, lbl: (b, i, 0, 0))
    return pl.pallas_call(
        _mamba_kernel,
        grid_spec=pltpu.PrefetchScalarGridSpec(
            num_scalar_prefetch=1,
            grid=(3, B),
            in_specs=[x_spec] + [pl.BlockSpec((1, 1) + a.shape[2:], im_w)
                                 for a in arrays],
            out_specs=x_spec,
            scratch_shapes=[pltpu.VMEM((SEG, D_STATE, D_INNER), jnp.float32),
                            pltpu.VMEM((SEG, D_STATE, D_INNER), jnp.float32),
                            pltpu.VMEM((SEG, D_STATE, D_INNER), jnp.float32)],
        ),
        out_shape=jax.ShapeDtypeStruct((B, 3, SEG, D_MODEL), jnp.float32),
    )(labels, feat4, *arrays)


def _attn_kernel(lbl_ref, x_ref, mask_ref, wq, bq, wk, bk, wv, bv, wp, bp,
                 g1, b1, o_ref):
    x = x_ref[0]
    ln = _ln(x, g1[0, 0], b1[0, 0])
    q = _dot(ln, wq[0]) + bq[0]
    k = _dot(ln, wk[0]) + bk[0]
    v = _dot(ln, wv[0]) + bv[0]
    mask = mask_ref[...]
    scale = 1.0 / math.sqrt(HD)
    outs = []
    for h in range(N_HEAD):
        sl = slice(h * HD, (h + 1) * HD)
        qh = q[:, sl]
        kh = k[:, sl]
        vh = v[:, sl]
        att = jax.lax.dot_general(qh, kh, (((1,), (1,)), ((), ())),
                                  preferred_element_type=jnp.float32) * scale
        att = jnp.where(mask, -1e30, att)
        att = att - jnp.max(att, axis=-1, keepdims=True)
        e = jnp.exp(att)
        att = e / jnp.sum(e, axis=-1, keepdims=True)
        outs.append(_dot(att, vh))
    y = jnp.concatenate(outs, axis=1)
    o_ref[0] = x + _dot(y, wp[0]) + bp[0]


def _attn_call(feat, aw, labels, mask):
    names = ['wq', 'bq', 'wk', 'bk', 'wv', 'bv', 'wp', 'bp', 'g1', 'b1']
    arrays = [aw[n] for n in names]
    im_w = lambda b, lbl: (lbl[b],) + (0,) * 2
    x_spec = pl.BlockSpec((1, TP, D_MODEL), lambda b, lbl: (b, 0, 0))
    return pl.pallas_call(
        _attn_kernel,
        grid_spec=pltpu.PrefetchScalarGridSpec(
            num_scalar_prefetch=1,
            grid=(B,),
            in_specs=[x_spec,
                      pl.BlockSpec((TP, TP), lambda b, lbl: (0, 0))]
                     + [pl.BlockSpec((1,) + a.shape[1:], im_w) for a in arrays],
            out_specs=x_spec,
        ),
        out_shape=jax.ShapeDtypeStruct((B, TP, D_MODEL), jnp.float32),
    )(labels, feat, mask, *arrays)


def _mlp_kernel(lbl_ref, x_ref, w1, b1, w2, b2, g2, bb2, o_ref):
    x = x_ref[0]
    ln = _ln(x, g2[0, 0], bb2[0, 0])
    a = _dot(ln, w1[0]) + b1[0]
    ge = 0.5 * a * (1.0 + jax.lax.erf(a * (1.0 / math.sqrt(2.0))))
    o_ref[0] = x + _dot(ge, w2[0]) + b2[0]


def _mlp_call(feat, mw, labels):
    names = ['w1', 'b1', 'w2', 'b2', 'g2', 'bb2']
    arrays = [mw[n] for n in names]
    im_w = lambda b, lbl: (lbl[b],) + (0,) * 2
    x_spec = pl.BlockSpec((1, TP, D_MODEL), lambda b, lbl: (b, 0, 0))
    return pl.pallas_call(
        _mlp_kernel,
        grid_spec=pltpu.PrefetchScalarGridSpec(
            num_scalar_prefetch=1,
            grid=(B,),
            in_specs=[x_spec] + [pl.BlockSpec((1,) + a.shape[1:], im_w)
                                 for a in arrays],
            out_specs=x_spec,
        ),
        out_shape=jax.ShapeDtypeStruct((B, TP, D_MODEL), jnp.float32),
    )(labels, feat, *arrays)


def _head_kernel(x_ref, t_ref, w1, b1, w2, b2, o_ref):
    b = pl.program_id(0)
    x = x_ref[0]
    h1 = _dot(x, w1[...]) + b1[...]
    logits = _dot(h1, w2[...]) + b2[...]
    m = jnp.max(logits, axis=-1, keepdims=True)
    lse = m + jnp.log(jnp.sum(jnp.exp(logits - m), axis=-1, keepdims=True))
    ls = logits - lse
    tgt = t_ref[0][:, :1]
    iota = jax.lax.broadcasted_iota(jnp.int32, (TP, OUT_DIM), 1)
    picked = jnp.sum(jnp.where(iota == tgt, ls, 0.0),
                     axis=(0, 1), keepdims=True)

    @pl.when(b == 0)
    def _():
        o_ref[...] = jnp.zeros((1, 1), jnp.float32)

    o_ref[...] = o_ref[...] - picked


def _head_call(feat, tgt_l, o1, o2):
    whole = lambda a: pl.BlockSpec(a.shape, lambda b: (0,) * a.ndim)
    b1 = o1['b'][None, :]
    b2 = o2['b'][None, :]
    return pl.pallas_call(
        _head_kernel,
        grid=(B,),
        in_specs=[
            pl.BlockSpec((1, TP, D_MODEL), lambda b: (b, 0, 0)),
            pl.BlockSpec((1, TP, 128), lambda b: (b, 0, 0)),
            whole(o1['w']), whole(b1), whole(o2['w']), whole(b2),
        ],
        out_specs=pl.BlockSpec((1, 1), lambda b: (0, 0)),
        out_shape=jax.ShapeDtypeStruct((1, 1), jnp.float32),
    )(feat, tgt_l, o1['w'], b1, o2['w'], b2)


def _stack_mamba(swabs):
    def S(f):
        return jnp.stack([jnp.stack([f(s['mamba'][i]) for s in swabs])
                          for i in range(3)])
    return {
        'win': S(lambda p: p['in_proj']['w']),
        'cw': S(lambda p: p['conv_w'].T),
        'cb': S(lambda p: p['conv_b'][None, :]),
        'wxdt': S(lambda p: p['x_proj']['w'][:, :DT_RANK]),
        'wxB': S(lambda p: p['x_proj']['w'][:, DT_RANK:DT_RANK + D_STATE]),
        'wxC': S(lambda p: p['x_proj']['w'][:, DT_RANK + D_STATE:]),
        'wdt': S(lambda p: p['dt_proj']['w']),
        'bdt': S(lambda p: p['dt_proj']['b'][None, :]),
        'alog': S(lambda p: p['A_log'].T),
        'dvec': S(lambda p: p['D'][None, :]),
        'wout': S(lambda p: p['out_proj']['w']),
        'g': jnp.stack([jnp.stack([s['mnorm'][i]['g'][None, :] for s in swabs])
                        for i in range(3)]),
        'b': jnp.stack([jnp.stack([s['mnorm'][i]['b'][None, :] for s in swabs])
                        for i in range(3)]),
    }


def _stack_attn(swabs):
    S = lambda f: jnp.stack([f(s) for s in swabs])
    return {
        'wq': S(lambda s: s['query']['w']), 'bq': S(lambda s: s['query']['b'][None, :]),
        'wk': S(lambda s: s['key']['w']), 'bk': S(lambda s: s['key']['b'][None, :]),
        'wv': S(lambda s: s['value']['w']), 'bv': S(lambda s: s['value']['b'][None, :]),
        'wp': S(lambda s: s['proj']['w']), 'bp': S(lambda s: s['proj']['b'][None, :]),
        'g1': S(lambda s: s['ln1']['g'][None, :]), 'b1': S(lambda s: s['ln1']['b'][None, :]),
    }


def _stack_mlp(swabs):
    S = lambda f: jnp.stack([f(s) for s in swabs])
    return {
        'w1': S(lambda s: s['mlp1']['w']), 'b1': S(lambda s: s['mlp1']['b'][None, :]),
        'w2': S(lambda s: s['mlp2']['w']), 'b2': S(lambda s: s['mlp2']['b'][None, :]),
        'g2': S(lambda s: s['ln2']['g'][None, :]), 'bb2': S(lambda s: s['ln2']['b'][None, :]),
    }


def _swab_calls(feat, swabs, labels, mask):
    feat4 = feat.reshape(B, 3, SEG, D_MODEL)
    feat4 = _mamba_call(feat4, _stack_mamba(swabs), labels)
    feat = feat4.reshape(B, TP, D_MODEL)
    feat = _attn_call(feat, _stack_attn(swabs), labels, mask)
    feat = _mlp_call(feat, _stack_mlp(swabs), labels)
    return feat


def kernel(music_librosa, pose_up, pose_down, label, params):
    label = label.astype(jnp.int32)
    order = jnp.argsort(label)
    lbl = label[order]
    mus = music_librosa[order][:, :TSRC, :]
    pu = pose_up[order].astype(jnp.int32)
    pd = pose_down[order].astype(jnp.int32)
    mus_p = jnp.pad(mus, ((0, 0), (0, SEG - TSRC), (0, 0)))
    pu_src = pu[:, :TSRC]
    pd_src = pd[:, :TSRC]
    neg3 = -jnp.ones((B, SEG - TSRC), jnp.int32)
    neg35 = -jnp.ones((B, SEG + SEG - TSRC), jnp.int32)
    tgt = jnp.concatenate([pu[:, 1:], neg3, pd[:, 1:], neg35], axis=1)
    tgt_l = jnp.broadcast_to(tgt[:, :, None], (B, TP, 128))
    zeros_lbl = jnp.zeros((B,), jnp.int32)
    mask = jnp.asarray(_MASK_NP)

    feat = _embed_call(pu_src, pd_src, params['pose_up_emb'],
                       params['pose_down_emb'], mus_p,
                       params['music1'], params['music2'])
    for lp in params['layers']:
        feat = _swab_calls(feat, [lp['global']], zeros_lbl, mask)
        feat = _swab_calls(feat, lp['experts'], lbl, mask)
    total = _head_call(feat, tgt_l, params['out1'], params['out2'])
    return total[0, 0] / (TSRC * B)


# speedup vs baseline: 7.1437x; 1.0028x over previous
import math

import numpy as np
import jax
import jax.numpy as jnp
from jax.experimental import pallas as pl
from jax.experimental.pallas import tpu as pltpu

D_MODEL = 512
D_STATE = 16
D_CONV = 4
D_INNER = 1024
DT_RANK = 32
N_HEAD = 8
HD = D_MODEL // N_HEAD
OUT_DIM = 4375
N_EXPERTS = 16
B = 32
TSRC = 29
SEG = 32
TP = 3 * SEG


def _build_padded_mask():
    ws, ts = 22, TSRC
    m = np.triu(np.ones((ts, ts), dtype=bool), 1)
    for i in range(ws, ts):
        m[i, :i - ws + 1] = True
    m87 = np.tile(m, (3, 3))
    mp = np.ones((TP, TP), dtype=bool)
    idx = np.concatenate([np.arange(ts), SEG + np.arange(ts), 2 * SEG + np.arange(ts)])
    mp[np.ix_(idx, idx)] = m87
    return mp


_MASK_NP = _build_padded_mask()


def _ln(x, g, b):
    mu = jnp.mean(x, axis=-1, keepdims=True)
    var = jnp.mean((x - mu) ** 2, axis=-1, keepdims=True)
    return (x - mu) * jax.lax.rsqrt(var + 1e-5) * g + b


def _dot(a, b):
    return jnp.dot(a, b, preferred_element_type=jnp.float32)


def _embed_kernel(pu_ref, pd_ref, up_tab, dn_tab, mus_ref,
                  m1w, m1b, m2w, m2b, o_ref):
    b = pl.program_id(0)
    o_ref[0] = jnp.zeros((TP, D_MODEL), jnp.float32)
    mus = mus_ref[0]
    h1 = _dot(mus, m1w[...]) + m1b[...]
    h2 = _dot(h1, m2w[...]) + m2b[...]
    o_ref[0, 2 * SEG:3 * SEG, :] = h2
    for l in range(TSRC):
        iu = pu_ref[b, l]
        idn = pd_ref[b, l]
        o_ref[0, l, :] = up_tab[iu, :]
        o_ref[0, SEG + l, :] = dn_tab[idn, :]


def _embed_call(pu_src, pd_src, up_tab, dn_tab, mus_p, m1, m2):
    whole = lambda a: pl.BlockSpec(a.shape, lambda b, pu, pd: (0,) * a.ndim)
    m1b = m1['b'][None, :]
    m2b = m2['b'][None, :]
    return pl.pallas_call(
        _embed_kernel,
        grid_spec=pltpu.PrefetchScalarGridSpec(
            num_scalar_prefetch=2,
            grid=(B,),
            in_specs=[
                whole(up_tab),
                whole(dn_tab),
                pl.BlockSpec((1, SEG, 35), lambda b, pu, pd: (b, 0, 0)),
                whole(m1['w']), whole(m1b), whole(m2['w']), whole(m2b),
            ],
            out_specs=pl.BlockSpec((1, TP, D_MODEL), lambda b, pu, pd: (b, 0, 0)),
        ),
        out_shape=jax.ShapeDtypeStruct((B, TP, D_MODEL), jnp.float32),
        compiler_params=pltpu.CompilerParams(
            dimension_semantics=("parallel",)),
    )(pu_src, pd_src, up_tab, dn_tab, mus_p, m1['w'], m1b, m2['w'], m2b)


def _mamba_kernel(lbl_ref, x_ref, win, cw, cb, wxdt, wxB, wxC, wdt, bdt,
                  alog, dvec, wout, gg, bb, o_ref, h_ref, dA_ref, dBu_ref):
    x = x_ref[0, 0]
    xz = _dot(x, win[0, 0])
    xr = xz[:, :D_INNER]
    z = xz[:, D_INNER:]
    cwm = cw[0, 0]
    zrow = jnp.zeros((1, D_INNER), jnp.float32)
    s1 = jnp.concatenate([zrow, xr[:SEG - 1]], axis=0)
    s2 = jnp.concatenate([jnp.zeros((2, D_INNER), jnp.float32), xr[:SEG - 2]], axis=0)
    s3 = jnp.concatenate([jnp.zeros((3, D_INNER), jnp.float32), xr[:SEG - 3]], axis=0)
    conv = (cb[0, 0] + xr * cwm[3:4, :] + s1 * cwm[2:3, :]
            + s2 * cwm[1:2, :] + s3 * cwm[0:1, :])
    xc = jax.nn.silu(conv)
    dt = _dot(xc, wxdt[0, 0])
    Bm = _dot(xc, wxB[0, 0])
    Cm = _dot(xc, wxC[0, 0])
    delta = jax.nn.softplus(_dot(dt, wdt[0, 0]) + bdt[0, 0])
    A_T = -jnp.exp(alog[0, 0])
    dA_ref[...] = jnp.exp(delta[:, None, :] * A_T[None])
    dBu_ref[...] = Bm[:, :, None] * (delta * xc)[:, None, :]
    h_ref[...] = jnp.zeros((SEG, D_STATE, D_INNER), jnp.float32)

    def body(l, h):
        da = dA_ref[pl.ds(l, 1)][0]
        db = dBu_ref[pl.ds(l, 1)][0]
        h = da * h + db
        h_ref[pl.ds(l, 1)] = h[None]
        return h

    jax.lax.fori_loop(0, TSRC, body, jnp.zeros((D_STATE, D_INNER), jnp.float32))
    ys = jnp.sum(h_ref[...] * Cm[:, :, None], axis=1)
    y = ys + xc * dvec[0, 0]
    y = y * jax.nn.silu(z)
    out = x + _dot(y, wout[0, 0])
    o_ref[0, 0] = _ln(out, gg[0, 0], bb[0, 0])


def _mamba_call(feat4, mw, labels):
    names = ['win', 'cw', 'cb', 'wxdt', 'wxB', 'wxC', 'wdt', 'bdt',
             'alog', 'dvec', 'wout', 'g', 'b']
    arrays = [mw[n] for n in names]
    im_w = lambda i, b, lbl: (i, lbl[b], 0, 0)
    x_spec = pl.BlockSpec((1, 1, SEG, D_MODEL), lambda i, b, lbl: (b, i, 0, 0))
    return pl.pallas_call(
        _mamba_kernel,
        grid_spec=pltpu.PrefetchScalarGridSpec(
            num_scalar_prefetch=1,
            grid=(3, B),
            in_specs=[x_spec] + [pl.BlockSpec((1, 1) + a.shape[2:], im_w)
                                 for a in arrays],
            out_specs=x_spec,
            scratch_shapes=[pltpu.VMEM((SEG, D_STATE, D_INNER), jnp.float32),
                            pltpu.VMEM((SEG, D_STATE, D_INNER), jnp.float32),
                            pltpu.VMEM((SEG, D_STATE, D_INNER), jnp.float32)],
        ),
        out_shape=jax.ShapeDtypeStruct((B, 3, SEG, D_MODEL), jnp.float32),
        compiler_params=pltpu.CompilerParams(
            dimension_semantics=("parallel", "arbitrary")),
    )(labels, feat4, *arrays)


def _attn_kernel(lbl_ref, x_ref, mask_ref, wq, bq, wk, bk, wv, bv, wp, bp,
                 g1, b1, o_ref):
    x = x_ref[0]
    ln = _ln(x, g1[0, 0], b1[0, 0])
    q = _dot(ln, wq[0]) + bq[0]
    k = _dot(ln, wk[0]) + bk[0]
    v = _dot(ln, wv[0]) + bv[0]
    mask = mask_ref[...]
    scale = 1.0 / math.sqrt(HD)
    outs = []
    for h in range(N_HEAD):
        sl = slice(h * HD, (h + 1) * HD)
        qh = q[:, sl]
        kh = k[:, sl]
        vh = v[:, sl]
        att = jax.lax.dot_general(qh, kh, (((1,), (1,)), ((), ())),
                                  preferred_element_type=jnp.float32) * scale
        att = jnp.where(mask, -1e30, att)
        att = att - jnp.max(att, axis=-1, keepdims=True)
        e = jnp.exp(att)
        att = e / jnp.sum(e, axis=-1, keepdims=True)
        outs.append(_dot(att, vh))
    y = jnp.concatenate(outs, axis=1)
    o_ref[0] = x + _dot(y, wp[0]) + bp[0]


def _attn_call(feat, aw, labels, mask):
    names = ['wq', 'bq', 'wk', 'bk', 'wv', 'bv', 'wp', 'bp', 'g1', 'b1']
    arrays = [aw[n] for n in names]
    im_w = lambda b, lbl: (lbl[b],) + (0,) * 2
    x_spec = pl.BlockSpec((1, TP, D_MODEL), lambda b, lbl: (b, 0, 0))
    return pl.pallas_call(
        _attn_kernel,
        grid_spec=pltpu.PrefetchScalarGridSpec(
            num_scalar_prefetch=1,
            grid=(B,),
            in_specs=[x_spec,
                      pl.BlockSpec((TP, TP), lambda b, lbl: (0, 0))]
                     + [pl.BlockSpec((1,) + a.shape[1:], im_w) for a in arrays],
            out_specs=x_spec,
        ),
        out_shape=jax.ShapeDtypeStruct((B, TP, D_MODEL), jnp.float32),
        compiler_params=pltpu.CompilerParams(
            dimension_semantics=("parallel",)),
    )(labels, feat, mask, *arrays)


def _mlp_kernel(lbl_ref, x_ref, w1, b1, w2, b2, g2, bb2, o_ref):
    x = x_ref[0]
    ln = _ln(x, g2[0, 0], bb2[0, 0])
    a = _dot(ln, w1[0]) + b1[0]
    ge = 0.5 * a * (1.0 + jax.lax.erf(a * (1.0 / math.sqrt(2.0))))
    o_ref[0] = x + _dot(ge, w2[0]) + b2[0]


def _mlp_call(feat, mw, labels):
    names = ['w1', 'b1', 'w2', 'b2', 'g2', 'bb2']
    arrays = [mw[n] for n in names]
    im_w = lambda b, lbl: (lbl[b],) + (0,) * 2
    x_spec = pl.BlockSpec((1, TP, D_MODEL), lambda b, lbl: (b, 0, 0))
    return pl.pallas_call(
        _mlp_kernel,
        grid_spec=pltpu.PrefetchScalarGridSpec(
            num_scalar_prefetch=1,
            grid=(B,),
            in_specs=[x_spec] + [pl.BlockSpec((1,) + a.shape[1:], im_w)
                                 for a in arrays],
            out_specs=x_spec,
        ),
        out_shape=jax.ShapeDtypeStruct((B, TP, D_MODEL), jnp.float32),
        compiler_params=pltpu.CompilerParams(
            dimension_semantics=("parallel",)),
    )(labels, feat, *arrays)


def _head_kernel(x_ref, t_ref, w1, b1, w2, b2, o_ref):
    b = pl.program_id(0)
    x = x_ref[0]
    h1 = _dot(x, w1[...]) + b1[...]
    logits = _dot(h1, w2[...]) + b2[...]
    m = jnp.max(logits, axis=-1, keepdims=True)
    lse = m + jnp.log(jnp.sum(jnp.exp(logits - m), axis=-1, keepdims=True))
    ls = logits - lse
    tgt = t_ref[0][:, :1]
    iota = jax.lax.broadcasted_iota(jnp.int32, (TP, OUT_DIM), 1)
    picked = jnp.sum(jnp.where(iota == tgt, ls, 0.0),
                     axis=(0, 1), keepdims=True)

    @pl.when(b == 0)
    def _():
        o_ref[...] = jnp.zeros((1, 1), jnp.float32)

    o_ref[...] = o_ref[...] - picked


def _head_call(feat, tgt_l, o1, o2):
    whole = lambda a: pl.BlockSpec(a.shape, lambda b: (0,) * a.ndim)
    b1 = o1['b'][None, :]
    b2 = o2['b'][None, :]
    return pl.pallas_call(
        _head_kernel,
        grid=(B,),
        in_specs=[
            pl.BlockSpec((1, TP, D_MODEL), lambda b: (b, 0, 0)),
            pl.BlockSpec((1, TP, 128), lambda b: (b, 0, 0)),
            whole(o1['w']), whole(b1), whole(o2['w']), whole(b2),
        ],
        out_specs=pl.BlockSpec((1, 1), lambda b: (0, 0)),
        out_shape=jax.ShapeDtypeStruct((1, 1), jnp.float32),
    )(feat, tgt_l, o1['w'], b1, o2['w'], b2)


def _stack_mamba(swabs):
    def S(f):
        return jnp.stack([jnp.stack([f(s['mamba'][i]) for s in swabs])
                          for i in range(3)])
    return {
        'win': S(lambda p: p['in_proj']['w']),
        'cw': S(lambda p: p['conv_w'].T),
        'cb': S(lambda p: p['conv_b'][None, :]),
        'wxdt': S(lambda p: p['x_proj']['w'][:, :DT_RANK]),
        'wxB': S(lambda p: p['x_proj']['w'][:, DT_RANK:DT_RANK + D_STATE]),
        'wxC': S(lambda p: p['x_proj']['w'][:, DT_RANK + D_STATE:]),
        'wdt': S(lambda p: p['dt_proj']['w']),
        'bdt': S(lambda p: p['dt_proj']['b'][None, :]),
        'alog': S(lambda p: p['A_log'].T),
        'dvec': S(lambda p: p['D'][None, :]),
        'wout': S(lambda p: p['out_proj']['w']),
        'g': jnp.stack([jnp.stack([s['mnorm'][i]['g'][None, :] for s in swabs])
                        for i in range(3)]),
        'b': jnp.stack([jnp.stack([s['mnorm'][i]['b'][None, :] for s in swabs])
                        for i in range(3)]),
    }


def _stack_attn(swabs):
    S = lambda f: jnp.stack([f(s) for s in swabs])
    return {
        'wq': S(lambda s: s['query']['w']), 'bq': S(lambda s: s['query']['b'][None, :]),
        'wk': S(lambda s: s['key']['w']), 'bk': S(lambda s: s['key']['b'][None, :]),
        'wv': S(lambda s: s['value']['w']), 'bv': S(lambda s: s['value']['b'][None, :]),
        'wp': S(lambda s: s['proj']['w']), 'bp': S(lambda s: s['proj']['b'][None, :]),
        'g1': S(lambda s: s['ln1']['g'][None, :]), 'b1': S(lambda s: s['ln1']['b'][None, :]),
    }


def _stack_mlp(swabs):
    S = lambda f: jnp.stack([f(s) for s in swabs])
    return {
        'w1': S(lambda s: s['mlp1']['w']), 'b1': S(lambda s: s['mlp1']['b'][None, :]),
        'w2': S(lambda s: s['mlp2']['w']), 'b2': S(lambda s: s['mlp2']['b'][None, :]),
        'g2': S(lambda s: s['ln2']['g'][None, :]), 'bb2': S(lambda s: s['ln2']['b'][None, :]),
    }


def _swab_calls(feat, swabs, labels, mask):
    feat4 = feat.reshape(B, 3, SEG, D_MODEL)
    feat4 = _mamba_call(feat4, _stack_mamba(swabs), labels)
    feat = feat4.reshape(B, TP, D_MODEL)
    feat = _attn_call(feat, _stack_attn(swabs), labels, mask)
    feat = _mlp_call(feat, _stack_mlp(swabs), labels)
    return feat


def kernel(music_librosa, pose_up, pose_down, label, params):
    label = label.astype(jnp.int32)
    order = jnp.argsort(label)
    lbl = label[order]
    mus = music_librosa[order][:, :TSRC, :]
    pu = pose_up[order].astype(jnp.int32)
    pd = pose_down[order].astype(jnp.int32)
    mus_p = jnp.pad(mus, ((0, 0), (0, SEG - TSRC), (0, 0)))
    pu_src = pu[:, :TSRC]
    pd_src = pd[:, :TSRC]
    neg3 = -jnp.ones((B, SEG - TSRC), jnp.int32)
    neg35 = -jnp.ones((B, SEG + SEG - TSRC), jnp.int32)
    tgt = jnp.concatenate([pu[:, 1:], neg3, pd[:, 1:], neg35], axis=1)
    tgt_l = jnp.broadcast_to(tgt[:, :, None], (B, TP, 128))
    zeros_lbl = jnp.zeros((B,), jnp.int32)
    mask = jnp.asarray(_MASK_NP)

    feat = _embed_call(pu_src, pd_src, params['pose_up_emb'],
                       params['pose_down_emb'], mus_p,
                       params['music1'], params['music2'])
    for lp in params['layers']:
        feat = _swab_calls(feat, [lp['global']], zeros_lbl, mask)
        feat = _swab_calls(feat, lp['experts'], lbl, mask)
    total = _head_call(feat, tgt_l, params['out1'], params['out2'])
    return total[0, 0] / (TSRC * B)


# speedup vs baseline: 7.8473x; 1.0985x over previous
import math

import numpy as np
import jax
import jax.numpy as jnp
from jax.experimental import pallas as pl
from jax.experimental.pallas import tpu as pltpu

D_MODEL = 512
D_STATE = 16
D_CONV = 4
D_INNER = 1024
DT_RANK = 32
N_HEAD = 8
HD = D_MODEL // N_HEAD
OUT_DIM = 4375
N_EXPERTS = 16
B = 32
TSRC = 29
SEG = 32
TP = 3 * SEG


def _build_padded_mask():
    ws, ts = 22, TSRC
    m = np.triu(np.ones((ts, ts), dtype=bool), 1)
    for i in range(ws, ts):
        m[i, :i - ws + 1] = True
    m87 = np.tile(m, (3, 3))
    mp = np.ones((TP, TP), dtype=bool)
    idx = np.concatenate([np.arange(ts), SEG + np.arange(ts), 2 * SEG + np.arange(ts)])
    mp[np.ix_(idx, idx)] = m87
    return mp


_MASK_NP = _build_padded_mask()


def _ln(x, g, b):
    mu = jnp.mean(x, axis=-1, keepdims=True)
    var = jnp.mean((x - mu) ** 2, axis=-1, keepdims=True)
    return (x - mu) * jax.lax.rsqrt(var + 1e-5) * g + b


def _dot(a, b):
    return jnp.dot(a, b, preferred_element_type=jnp.float32)


def _embed_kernel(pu_ref, pd_ref, up_tab, dn_tab, mus_ref,
                  m1w, m1b, m2w, m2b, o_ref):
    b = pl.program_id(0)
    o_ref[0] = jnp.zeros((TP, D_MODEL), jnp.float32)
    mus = mus_ref[0]
    h1 = _dot(mus, m1w[...]) + m1b[...]
    h2 = _dot(h1, m2w[...]) + m2b[...]
    o_ref[0, 2 * SEG:3 * SEG, :] = h2
    for l in range(TSRC):
        iu = pu_ref[b, l]
        idn = pd_ref[b, l]
        o_ref[0, l, :] = up_tab[iu, :]
        o_ref[0, SEG + l, :] = dn_tab[idn, :]


def _embed_call(pu_src, pd_src, up_tab, dn_tab, mus_p, m1, m2):
    whole = lambda a: pl.BlockSpec(a.shape, lambda b, pu, pd: (0,) * a.ndim)
    m1b = m1['b'][None, :]
    m2b = m2['b'][None, :]
    return pl.pallas_call(
        _embed_kernel,
        grid_spec=pltpu.PrefetchScalarGridSpec(
            num_scalar_prefetch=2,
            grid=(B,),
            in_specs=[
                whole(up_tab),
                whole(dn_tab),
                pl.BlockSpec((1, SEG, 35), lambda b, pu, pd: (b, 0, 0)),
                whole(m1['w']), whole(m1b), whole(m2['w']), whole(m2b),
            ],
            out_specs=pl.BlockSpec((1, TP, D_MODEL), lambda b, pu, pd: (b, 0, 0)),
        ),
        out_shape=jax.ShapeDtypeStruct((B, TP, D_MODEL), jnp.float32),
        compiler_params=pltpu.CompilerParams(
            dimension_semantics=("parallel",)),
    )(pu_src, pd_src, up_tab, dn_tab, mus_p, m1['w'], m1b, m2['w'], m2b)


def _mamba_kernel(lbl_ref, x_ref, win, cw, cb, wxdt, wxB, wxC, wdt, bdt,
                  alog, dvec, wout, gg, bb, o_ref, h_ref, dA_ref, dBu_ref):
    x = x_ref[0, 0]
    xz = _dot(x, win[0, 0])
    xr = xz[:, :D_INNER]
    z = xz[:, D_INNER:]
    cwm = cw[0, 0]
    zrow = jnp.zeros((1, D_INNER), jnp.float32)
    s1 = jnp.concatenate([zrow, xr[:SEG - 1]], axis=0)
    s2 = jnp.concatenate([jnp.zeros((2, D_INNER), jnp.float32), xr[:SEG - 2]], axis=0)
    s3 = jnp.concatenate([jnp.zeros((3, D_INNER), jnp.float32), xr[:SEG - 3]], axis=0)
    conv = (cb[0, 0] + xr * cwm[3:4, :] + s1 * cwm[2:3, :]
            + s2 * cwm[1:2, :] + s3 * cwm[0:1, :])
    xc = jax.nn.silu(conv)
    dt = _dot(xc, wxdt[0, 0])
    Bm = _dot(xc, wxB[0, 0])
    Cm = _dot(xc, wxC[0, 0])
    delta = jax.nn.softplus(_dot(dt, wdt[0, 0]) + bdt[0, 0])
    A_T = -jnp.exp(alog[0, 0])
    dA_ref[...] = jnp.exp(delta[:, None, :] * A_T[None])
    dBu_ref[...] = Bm[:, :, None] * (delta * xc)[:, None, :]
    h_ref[...] = jnp.zeros((SEG, D_STATE, D_INNER), jnp.float32)

    def body(l, h):
        da = dA_ref[pl.ds(l, 1)][0]
        db = dBu_ref[pl.ds(l, 1)][0]
        h = da * h + db
        h_ref[pl.ds(l, 1)] = h[None]
        return h

    jax.lax.fori_loop(0, TSRC, body, jnp.zeros((D_STATE, D_INNER), jnp.float32))
    ys = jnp.sum(h_ref[...] * Cm[:, :, None], axis=1)
    y = ys + xc * dvec[0, 0]
    y = y * jax.nn.silu(z)
    out = x + _dot(y, wout[0, 0])
    o_ref[0, 0] = _ln(out, gg[0, 0], bb[0, 0])


def _mamba_call(feat4, mw, labels):
    names = ['win', 'cw', 'cb', 'wxdt', 'wxB', 'wxC', 'wdt', 'bdt',
             'alog', 'dvec', 'wout', 'g', 'b']
    arrays = [mw[n] for n in names]
    im_w = lambda i, b, lbl: (i, lbl[b], 0, 0)
    x_spec = pl.BlockSpec((1, 1, SEG, D_MODEL), lambda i, b, lbl: (b, i, 0, 0))
    return pl.pallas_call(
        _mamba_kernel,
        grid_spec=pltpu.PrefetchScalarGridSpec(
            num_scalar_prefetch=1,
            grid=(3, B),
            in_specs=[x_spec] + [pl.BlockSpec((1, 1) + a.shape[2:], im_w)
                                 for a in arrays],
            out_specs=x_spec,
            scratch_shapes=[pltpu.VMEM((SEG, D_STATE, D_INNER), jnp.float32),
                            pltpu.VMEM((SEG, D_STATE, D_INNER), jnp.float32),
                            pltpu.VMEM((SEG, D_STATE, D_INNER), jnp.float32)],
        ),
        out_shape=jax.ShapeDtypeStruct((B, 3, SEG, D_MODEL), jnp.float32),
        compiler_params=pltpu.CompilerParams(
            dimension_semantics=("parallel", "arbitrary")),
    )(labels, feat4, *arrays)


def _attn_kernel(lbl_ref, x_ref, mask_ref, wq, bq, wk, bk, wv, bv, wp, bp,
                 g1, b1, o_ref):
    x = x_ref[0]
    ln = _ln(x, g1[0, 0], b1[0, 0])
    q = _dot(ln, wq[0]) + bq[0]
    k = _dot(ln, wk[0]) + bk[0]
    v = _dot(ln, wv[0]) + bv[0]
    mask = mask_ref[...]
    scale = 1.0 / math.sqrt(HD)
    outs = []
    for h in range(N_HEAD):
        sl = slice(h * HD, (h + 1) * HD)
        qh = q[:, sl]
        kh = k[:, sl]
        vh = v[:, sl]
        att = jax.lax.dot_general(qh, kh, (((1,), (1,)), ((), ())),
                                  preferred_element_type=jnp.float32) * scale
        att = jnp.where(mask, -1e30, att)
        att = att - jnp.max(att, axis=-1, keepdims=True)
        e = jnp.exp(att)
        att = e / jnp.sum(e, axis=-1, keepdims=True)
        outs.append(_dot(att, vh))
    y = jnp.concatenate(outs, axis=1)
    o_ref[0] = x + _dot(y, wp[0]) + bp[0]


def _attn_call(feat, aw, labels, mask):
    names = ['wq', 'bq', 'wk', 'bk', 'wv', 'bv', 'wp', 'bp', 'g1', 'b1']
    arrays = [aw[n] for n in names]
    im_w = lambda b, lbl: (lbl[b],) + (0,) * 2
    x_spec = pl.BlockSpec((1, TP, D_MODEL), lambda b, lbl: (b, 0, 0))
    return pl.pallas_call(
        _attn_kernel,
        grid_spec=pltpu.PrefetchScalarGridSpec(
            num_scalar_prefetch=1,
            grid=(B,),
            in_specs=[x_spec,
                      pl.BlockSpec((TP, TP), lambda b, lbl: (0, 0))]
                     + [pl.BlockSpec((1,) + a.shape[1:], im_w) for a in arrays],
            out_specs=x_spec,
        ),
        out_shape=jax.ShapeDtypeStruct((B, TP, D_MODEL), jnp.float32),
        compiler_params=pltpu.CompilerParams(
            dimension_semantics=("parallel",)),
    )(labels, feat, mask, *arrays)


def _mlp_kernel(lbl_ref, x_ref, w1, b1, w2, b2, g2, bb2, o_ref):
    x = x_ref[0]
    ln = _ln(x, g2[0, 0], bb2[0, 0])
    a = _dot(ln, w1[0]) + b1[0]
    ge = 0.5 * a * (1.0 + jax.lax.erf(a * (1.0 / math.sqrt(2.0))))
    o_ref[0] = x + _dot(ge, w2[0]) + b2[0]


def _mlp_call(feat, mw, labels):
    names = ['w1', 'b1', 'w2', 'b2', 'g2', 'bb2']
    arrays = [mw[n] for n in names]
    im_w = lambda b, lbl: (lbl[b],) + (0,) * 2
    x_spec = pl.BlockSpec((1, TP, D_MODEL), lambda b, lbl: (b, 0, 0))
    return pl.pallas_call(
        _mlp_kernel,
        grid_spec=pltpu.PrefetchScalarGridSpec(
            num_scalar_prefetch=1,
            grid=(B,),
            in_specs=[x_spec] + [pl.BlockSpec((1,) + a.shape[1:], im_w)
                                 for a in arrays],
            out_specs=x_spec,
        ),
        out_shape=jax.ShapeDtypeStruct((B, TP, D_MODEL), jnp.float32),
        compiler_params=pltpu.CompilerParams(
            dimension_semantics=("parallel",)),
    )(labels, feat, *arrays)


MCH = 4
ACH = 8


def _gmamba_kernel(x_ref, win, cw, cb, wxdt, wxB, wxC, wdt, bdt,
                   alog, dvec, wout, gg, bb, o_ref, h_ref, dA_ref, dBu_ref):
    x = x_ref[:, 0]
    x2 = x.reshape(MCH * SEG, D_MODEL)
    xz = _dot(x2, win[0])
    xr = xz[:, :D_INNER].reshape(MCH, SEG, D_INNER)
    z = xz[:, D_INNER:].reshape(MCH, SEG, D_INNER)
    cwm = cw[0]
    s1 = jnp.concatenate([jnp.zeros((MCH, 1, D_INNER), jnp.float32),
                          xr[:, :SEG - 1]], axis=1)
    s2 = jnp.concatenate([jnp.zeros((MCH, 2, D_INNER), jnp.float32),
                          xr[:, :SEG - 2]], axis=1)
    s3 = jnp.concatenate([jnp.zeros((MCH, 3, D_INNER), jnp.float32),
                          xr[:, :SEG - 3]], axis=1)
    conv = (cb[0] + xr * cwm[3:4, :] + s1 * cwm[2:3, :]
            + s2 * cwm[1:2, :] + s3 * cwm[0:1, :])
    xc = jax.nn.silu(conv)
    xc2 = xc.reshape(MCH * SEG, D_INNER)
    dt = _dot(xc2, wxdt[0])
    Bm = _dot(xc2, wxB[0]).reshape(MCH, SEG, D_STATE)
    Cm = _dot(xc2, wxC[0]).reshape(MCH, SEG, D_STATE)
    delta = jax.nn.softplus(_dot(dt, wdt[0]) + bdt[0]
                            ).reshape(MCH, SEG, D_INNER)
    A_T = -jnp.exp(alog[0])
    dA_ref[...] = jnp.exp(delta[:, :, None, :] * A_T[None, None])
    dBu_ref[...] = Bm[:, :, :, None] * (delta * xc)[:, :, None, :]
    h_ref[...] = jnp.zeros((MCH, SEG, D_STATE, D_INNER), jnp.float32)

    def body(l, h):
        da = dA_ref[:, pl.ds(l, 1)][:, 0]
        db = dBu_ref[:, pl.ds(l, 1)][:, 0]
        h = da * h + db
        h_ref[:, pl.ds(l, 1)] = h[:, None]
        return h

    jax.lax.fori_loop(0, TSRC, body,
                      jnp.zeros((MCH, D_STATE, D_INNER), jnp.float32))
    ys = jnp.sum(h_ref[...] * Cm[:, :, :, None], axis=2)
    y = ys + xc * dvec[0]
    y = y * jax.nn.silu(z)
    out = x + _dot(y.reshape(MCH * SEG, D_INNER),
                   wout[0]).reshape(MCH, SEG, D_MODEL)
    o_ref[:, 0] = _ln(out, gg[0], bb[0])


def _gmamba_call(feat4, mw):
    names = ['win', 'cw', 'cb', 'wxdt', 'wxB', 'wxC', 'wdt', 'bdt',
             'alog', 'dvec', 'wout', 'g', 'b']
    arrays = [mw[n][:, 0] for n in names]
    x_spec = pl.BlockSpec((MCH, 1, SEG, D_MODEL), lambda i, c: (c, i, 0, 0))

    def wspec(a):
        nd = a.ndim
        return pl.BlockSpec((1,) + a.shape[1:],
                            lambda i, c, _nd=nd: (i,) + (0,) * (_nd - 1))

    return pl.pallas_call(
        _gmamba_kernel,
        grid=(3, B // MCH),
        in_specs=[x_spec] + [wspec(a) for a in arrays],
        out_specs=x_spec,
        out_shape=jax.ShapeDtypeStruct((B, 3, SEG, D_MODEL), jnp.float32),
        scratch_shapes=[pltpu.VMEM((MCH, SEG, D_STATE, D_INNER), jnp.float32),
                        pltpu.VMEM((MCH, SEG, D_STATE, D_INNER), jnp.float32),
                        pltpu.VMEM((MCH, SEG, D_STATE, D_INNER), jnp.float32)],
        compiler_params=pltpu.CompilerParams(
            dimension_semantics=("arbitrary", "parallel")),
    )(feat4, *arrays)


def _gattn_kernel(x_ref, mask_ref, wq, bq, wk, bk, wv, bv, wp, bp,
                  g1, b1, o_ref):
    x = x_ref[...]
    ln = _ln(x, g1[...], b1[...])
    ln2 = ln.reshape(ACH * TP, D_MODEL)
    q = (_dot(ln2, wq[...]) + bq[...]).reshape(ACH, TP, D_MODEL)
    k = (_dot(ln2, wk[...]) + bk[...]).reshape(ACH, TP, D_MODEL)
    v = (_dot(ln2, wv[...]) + bv[...]).reshape(ACH, TP, D_MODEL)
    mask = mask_ref[...]
    scale = 1.0 / math.sqrt(HD)
    outs = []
    for h in range(N_HEAD):
        sl = slice(h * HD, (h + 1) * HD)
        qh = q[:, :, sl]
        kh = k[:, :, sl]
        vh = v[:, :, sl]
        att = jax.lax.dot_general(qh, kh, (((2,), (2,)), ((0,), (0,))),
                                  preferred_element_type=jnp.float32) * scale
        att = jnp.where(mask[None], -1e30, att)
        att = att - jnp.max(att, axis=-1, keepdims=True)
        e = jnp.exp(att)
        att = e / jnp.sum(e, axis=-1, keepdims=True)
        outs.append(jax.lax.dot_general(att, vh, (((2,), (1,)), ((0,), (0,))),
                                        preferred_element_type=jnp.float32))
    y = jnp.concatenate(outs, axis=2).reshape(ACH * TP, D_MODEL)
    o_ref[...] = x + (_dot(y, wp[...]) + bp[...]).reshape(ACH, TP, D_MODEL)


def _gattn_call(feat, aw, mask):
    names = ['wq', 'bq', 'wk', 'bk', 'wv', 'bv', 'wp', 'bp', 'g1', 'b1']
    arrays = [aw[n][0] for n in names]
    x_spec = pl.BlockSpec((ACH, TP, D_MODEL), lambda c: (c, 0, 0))
    return pl.pallas_call(
        _gattn_kernel,
        grid=(B // ACH,),
        in_specs=[x_spec, pl.BlockSpec((TP, TP), lambda c: (0, 0))]
                 + [pl.BlockSpec(a.shape, lambda c: (0,) * a.ndim)
                    for a in arrays],
        out_specs=x_spec,
        out_shape=jax.ShapeDtypeStruct((B, TP, D_MODEL), jnp.float32),
        compiler_params=pltpu.CompilerParams(
            dimension_semantics=("parallel",)),
    )(feat, mask, *arrays)


def _gmlp_kernel(x_ref, w1, b1, w2, b2, g2, bb2, o_ref):
    x = x_ref[...]
    ln = _ln(x, g2[...], bb2[...]).reshape(ACH * TP, D_MODEL)
    a = _dot(ln, w1[...]) + b1[...]
    ge = 0.5 * a * (1.0 + jax.lax.erf(a * (1.0 / math.sqrt(2.0))))
    o_ref[...] = x + (_dot(ge, w2[...]) + b2[...]).reshape(ACH, TP, D_MODEL)


def _gmlp_call(feat, mw):
    names = ['w1', 'b1', 'w2', 'b2', 'g2', 'bb2']
    arrays = [mw[n][0] for n in names]
    x_spec = pl.BlockSpec((ACH, TP, D_MODEL), lambda c: (c, 0, 0))
    return pl.pallas_call(
        _gmlp_kernel,
        grid=(B // ACH,),
        in_specs=[x_spec] + [pl.BlockSpec(a.shape, lambda c: (0,) * a.ndim)
                             for a in arrays],
        out_specs=x_spec,
        out_shape=jax.ShapeDtypeStruct((B, TP, D_MODEL), jnp.float32),
        compiler_params=pltpu.CompilerParams(
            dimension_semantics=("parallel",)),
    )(feat, *arrays)


def _head_kernel(x_ref, t_ref, w1, b1, w2, b2, o_ref):
    b = pl.program_id(0)
    x = x_ref[0]
    h1 = _dot(x, w1[...]) + b1[...]
    logits = _dot(h1, w2[...]) + b2[...]
    m = jnp.max(logits, axis=-1, keepdims=True)
    lse = m + jnp.log(jnp.sum(jnp.exp(logits - m), axis=-1, keepdims=True))
    ls = logits - lse
    tgt = t_ref[0][:, :1]
    iota = jax.lax.broadcasted_iota(jnp.int32, (TP, OUT_DIM), 1)
    picked = jnp.sum(jnp.where(iota == tgt, ls, 0.0),
                     axis=(0, 1), keepdims=True)

    @pl.when(b == 0)
    def _():
        o_ref[...] = jnp.zeros((1, 1), jnp.float32)

    o_ref[...] = o_ref[...] - picked


def _head_call(feat, tgt_l, o1, o2):
    whole = lambda a: pl.BlockSpec(a.shape, lambda b: (0,) * a.ndim)
    b1 = o1['b'][None, :]
    b2 = o2['b'][None, :]
    return pl.pallas_call(
        _head_kernel,
        grid=(B,),
        in_specs=[
            pl.BlockSpec((1, TP, D_MODEL), lambda b: (b, 0, 0)),
            pl.BlockSpec((1, TP, 128), lambda b: (b, 0, 0)),
            whole(o1['w']), whole(b1), whole(o2['w']), whole(b2),
        ],
        out_specs=pl.BlockSpec((1, 1), lambda b: (0, 0)),
        out_shape=jax.ShapeDtypeStruct((1, 1), jnp.float32),
    )(feat, tgt_l, o1['w'], b1, o2['w'], b2)


def _stack_mamba(swabs):
    def S(f):
        return jnp.stack([jnp.stack([f(s['mamba'][i]) for s in swabs])
                          for i in range(3)])
    return {
        'win': S(lambda p: p['in_proj']['w']),
        'cw': S(lambda p: p['conv_w'].T),
        'cb': S(lambda p: p['conv_b'][None, :]),
        'wxdt': S(lambda p: p['x_proj']['w'][:, :DT_RANK]),
        'wxB': S(lambda p: p['x_proj']['w'][:, DT_RANK:DT_RANK + D_STATE]),
        'wxC': S(lambda p: p['x_proj']['w'][:, DT_RANK + D_STATE:]),
        'wdt': S(lambda p: p['dt_proj']['w']),
        'bdt': S(lambda p: p['dt_proj']['b'][None, :]),
        'alog': S(lambda p: p['A_log'].T),
        'dvec': S(lambda p: p['D'][None, :]),
        'wout': S(lambda p: p['out_proj']['w']),
        'g': jnp.stack([jnp.stack([s['mnorm'][i]['g'][None, :] for s in swabs])
                        for i in range(3)]),
        'b': jnp.stack([jnp.stack([s['mnorm'][i]['b'][None, :] for s in swabs])
                        for i in range(3)]),
    }


def _stack_attn(swabs):
    S = lambda f: jnp.stack([f(s) for s in swabs])
    return {
        'wq': S(lambda s: s['query']['w']), 'bq': S(lambda s: s['query']['b'][None, :]),
        'wk': S(lambda s: s['key']['w']), 'bk': S(lambda s: s['key']['b'][None, :]),
        'wv': S(lambda s: s['value']['w']), 'bv': S(lambda s: s['value']['b'][None, :]),
        'wp': S(lambda s: s['proj']['w']), 'bp': S(lambda s: s['proj']['b'][None, :]),
        'g1': S(lambda s: s['ln1']['g'][None, :]), 'b1': S(lambda s: s['ln1']['b'][None, :]),
    }


def _stack_mlp(swabs):
    S = lambda f: jnp.stack([f(s) for s in swabs])
    return {
        'w1': S(lambda s: s['mlp1']['w']), 'b1': S(lambda s: s['mlp1']['b'][None, :]),
        'w2': S(lambda s: s['mlp2']['w']), 'b2': S(lambda s: s['mlp2']['b'][None, :]),
        'g2': S(lambda s: s['ln2']['g'][None, :]), 'bb2': S(lambda s: s['ln2']['b'][None, :]),
    }


def _swab_calls(feat, swabs, labels, mask):
    feat4 = feat.reshape(B, 3, SEG, D_MODEL)
    feat4 = _mamba_call(feat4, _stack_mamba(swabs), labels)
    feat = feat4.reshape(B, TP, D_MODEL)
    feat = _attn_call(feat, _stack_attn(swabs), labels, mask)
    feat = _mlp_call(feat, _stack_mlp(swabs), labels)
    return feat


def _gswab_calls(feat, swab, mask):
    feat4 = feat.reshape(B, 3, SEG, D_MODEL)
    feat4 = _gmamba_call(feat4, _stack_mamba([swab]))
    feat = feat4.reshape(B, TP, D_MODEL)
    feat = _gattn_call(feat, _stack_attn([swab]), mask)
    feat = _gmlp_call(feat, _stack_mlp([swab]))
    return feat


def kernel(music_librosa, pose_up, pose_down, label, params):
    label = label.astype(jnp.int32)
    order = jnp.argsort(label)
    lbl = label[order]
    mus = music_librosa[order][:, :TSRC, :]
    pu = pose_up[order].astype(jnp.int32)
    pd = pose_down[order].astype(jnp.int32)
    mus_p = jnp.pad(mus, ((0, 0), (0, SEG - TSRC), (0, 0)))
    pu_src = pu[:, :TSRC]
    pd_src = pd[:, :TSRC]
    neg3 = -jnp.ones((B, SEG - TSRC), jnp.int32)
    neg35 = -jnp.ones((B, SEG + SEG - TSRC), jnp.int32)
    tgt = jnp.concatenate([pu[:, 1:], neg3, pd[:, 1:], neg35], axis=1)
    tgt_l = jnp.broadcast_to(tgt[:, :, None], (B, TP, 128))
    zeros_lbl = jnp.zeros((B,), jnp.int32)
    mask = jnp.asarray(_MASK_NP)

    feat = _embed_call(pu_src, pd_src, params['pose_up_emb'],
                       params['pose_down_emb'], mus_p,
                       params['music1'], params['music2'])
    for lp in params['layers']:
        feat = _gswab_calls(feat, lp['global'], mask)
        feat = _swab_calls(feat, lp['experts'], lbl, mask)
    total = _head_call(feat, tgt_l, params['out1'], params['out2'])
    return total[0, 0] / (TSRC * B)
